# baseline (speedup 1.0000x reference)
import jax
import jax.numpy as jnp
from jax.experimental import pallas as pl
from jax.experimental.pallas import tpu as pltpu


def _body(lens_ref, x_ref, codes_ref, W_ref, bias_ref, u_ref, out_ref):
    b_id = pl.program_id(0)
    xb = x_ref[0]
    s = jax.nn.sigmoid(xb)
    h = jnp.tanh(
        jnp.dot(s, W_ref[...], preferred_element_type=jnp.float32)
        + bias_ref[...]
    )
    vu = jnp.sum(h * u_ref[...], axis=1, keepdims=True)
    T = xb.shape[0]
    t_iota = jax.lax.broadcasted_iota(jnp.int32, (T, 1), 0)
    vu = jnp.where(t_iota < lens_ref[b_id], vu, -1e9)
    m = jnp.max(vu, axis=0, keepdims=True)
    e = jnp.exp(vu - m)
    score = e / jnp.sum(e, axis=0, keepdims=True)

    codes = codes_ref[0]
    c_iota = jax.lax.broadcasted_iota(jnp.int32, xb.shape, 1)
    onehot = c_iota == codes
    x_g = jnp.sum(jnp.where(onehot, xb, 0.0), axis=1, keepdims=True)
    val = jax.nn.sigmoid(x_g + score)
    out_ref[0] = jnp.where(onehot, val, s)


def kernel(x, lens, target_codes, W, b, u):
    B, T, C = x.shape
    A = W.shape[1]
    codes3 = target_codes.reshape(B, T, 1)
    bias2 = b.reshape(1, A)
    u2 = u.reshape(1, A)
    return pl.pallas_call(
        _body,
        grid=(B,),
        in_specs=[
            pl.BlockSpec(memory_space=pltpu.SMEM),
            pl.BlockSpec((1, T, C), lambda i: (i, 0, 0)),
            pl.BlockSpec((1, T, 1), lambda i: (i, 0, 0)),
            pl.BlockSpec((C, A), lambda i: (0, 0)),
            pl.BlockSpec((1, A), lambda i: (0, 0)),
            pl.BlockSpec((1, A), lambda i: (0, 0)),
        ],
        out_specs=pl.BlockSpec((1, T, C), lambda i: (i, 0, 0)),
        out_shape=jax.ShapeDtypeStruct((B, T, C), jnp.float32),
        compiler_params=pltpu.CompilerParams(
            dimension_semantics=("parallel",),
        ),
    )(lens, x, codes3, W, bias2, u2)


# speedup vs baseline: 1.0361x; 1.0361x over previous
import jax
import jax.numpy as jnp
from jax.experimental import pallas as pl
from jax.experimental.pallas import tpu as pltpu


def _body(lens_ref, x_ref, codes_ref, W_ref, bias_ref, u_ref, out_ref):
    b_id = pl.program_id(0)
    xb = x_ref[0]
    s = jnp.tanh(xb * 0.5) * 0.5 + 0.5
    h = jnp.tanh(
        jnp.dot(s, W_ref[...], preferred_element_type=jnp.float32)
        + bias_ref[...]
    )
    vu = jnp.sum(h * u_ref[...], axis=1, keepdims=True)
    T = xb.shape[0]
    t_iota = jax.lax.broadcasted_iota(jnp.int32, (T, 1), 0)
    vu = jnp.where(t_iota < lens_ref[b_id], vu, -1e9)
    m = jnp.max(vu, axis=0, keepdims=True)
    e = jnp.exp(vu - m)
    score = e / jnp.sum(e, axis=0, keepdims=True)

    codes = codes_ref[0]
    c_iota = jax.lax.broadcasted_iota(jnp.int32, xb.shape, 1)
    onehot = c_iota == codes
    x_g = jnp.sum(jnp.where(onehot, xb, 0.0), axis=1, keepdims=True)
    val = jax.nn.sigmoid(x_g + score)
    out_ref[0] = jnp.where(onehot, val, s)


def kernel(x, lens, target_codes, W, b, u):
    B, T, C = x.shape
    A = W.shape[1]
    codes3 = target_codes.reshape(B, T, 1)
    bias2 = b.reshape(1, A)
    u2 = u.reshape(1, A)
    return pl.pallas_call(
        _body,
        grid=(B,),
        in_specs=[
            pl.BlockSpec(memory_space=pltpu.SMEM),
            pl.BlockSpec((1, T, C), lambda i: (i, 0, 0)),
            pl.BlockSpec((1, T, 1), lambda i: (i, 0, 0)),
            pl.BlockSpec((C, A), lambda i: (0, 0)),
            pl.BlockSpec((1, A), lambda i: (0, 0)),
            pl.BlockSpec((1, A), lambda i: (0, 0)),
        ],
        out_specs=pl.BlockSpec((1, T, C), lambda i: (i, 0, 0)),
        out_shape=jax.ShapeDtypeStruct((B, T, C), jnp.float32),
        compiler_params=pltpu.CompilerParams(
            dimension_semantics=("parallel",),
        ),
    )(lens, x, codes3, W, bias2, u2)


# speedup vs baseline: 1.0368x; 1.0006x over previous
import jax
import jax.numpy as jnp
from jax.experimental import pallas as pl
from jax.experimental.pallas import tpu as pltpu


def _body(lens_ref, x_ref, codes_ref, W_ref, bias_ref, u_ref, out_ref):
    b_id = pl.program_id(0)
    xb = x_ref[0]
    s = jnp.tanh(xb * 0.5) * 0.5 + 0.5
    codes = codes_ref[0]
    c_iota = jax.lax.broadcasted_iota(jnp.int32, xb.shape, 1)
    onehot = c_iota == codes
    x_g = jnp.sum(jnp.where(onehot, xb, 0.0), axis=1, keepdims=True)
    h = jnp.tanh(
        jnp.dot(s, W_ref[...], preferred_element_type=jnp.float32)
        + bias_ref[...]
    )
    vu = jnp.sum(h * u_ref[...], axis=1, keepdims=True)
    T = xb.shape[0]
    t_iota = jax.lax.broadcasted_iota(jnp.int32, (T, 1), 0)
    vu = jnp.where(t_iota < lens_ref[b_id], vu, -1e9)
    m = jnp.max(vu, axis=0, keepdims=True)
    e = jnp.exp(vu - m)
    score = e / jnp.sum(e, axis=0, keepdims=True)

    val = jax.nn.sigmoid(x_g + score)
    out_ref[0] = jnp.where(onehot, val, s)


def kernel(x, lens, target_codes, W, b, u):
    B, T, C = x.shape
    A = W.shape[1]
    codes3 = target_codes.reshape(B, T, 1)
    bias2 = b.reshape(1, A)
    u2 = u.reshape(1, A)
    return pl.pallas_call(
        _body,
        grid=(B,),
        in_specs=[
            pl.BlockSpec(memory_space=pltpu.SMEM),
            pl.BlockSpec((1, T, C), lambda i: (i, 0, 0)),
            pl.BlockSpec((1, T, 1), lambda i: (i, 0, 0)),
            pl.BlockSpec((C, A), lambda i: (0, 0)),
            pl.BlockSpec((1, A), lambda i: (0, 0)),
            pl.BlockSpec((1, A), lambda i: (0, 0)),
        ],
        out_specs=pl.BlockSpec((1, T, C), lambda i: (i, 0, 0)),
        out_shape=jax.ShapeDtypeStruct((B, T, C), jnp.float32),
        compiler_params=pltpu.CompilerParams(
            dimension_semantics=("parallel",),
        ),
    )(lens, x, codes3, W, bias2, u2)
